# Initial kernel scaffold; baseline (speedup 1.0000x reference)
import jax
import jax.numpy as jnp
from jax.experimental import pallas as pl


def kernel(walks, sample_node, emb_table):
    raise NotImplementedError("write your pallas kernel here")


# speedup vs baseline: 1.7234x; 1.7234x over previous
import jax
import jax.numpy as jnp
from jax import lax
from jax.experimental import pallas as pl
from jax.experimental.pallas import tpu as pltpu
from jax.experimental.pallas import tpu_sc as plsc

NC = 2
NS = 16
NW = NC * NS
PV = 1 << 20
B = 16384
WL = 50
KNEG = 5
D = 64
IDXW = WL + KNEG + 1
ROWS_PER_TILE = B // NW
PRES_PER_TILE = PV // NW

def _mk_mesh():
    return plsc.VectorSubcoreMesh(core_axis_name="c", subcore_axis_name="s")


def _wid():
    return lax.axis_index("s") * NC + lax.axis_index("c")


def _presence_body(start_hbm, pres_hbm, idx_v, buf_v):
    lo = _wid() * PRES_PER_TILE
    pltpu.sync_copy(start_hbm, idx_v)
    zeros = jnp.zeros((16,), jnp.float32)
    ones = jnp.ones((16,), jnp.float32)

    def zero_body(i, _):
        buf_v[pl.ds(i * 16, 16)] = zeros
        return 0

    lax.fori_loop(0, PRES_PER_TILE // 16, zero_body, 0)

    def scat_body(i, _):
        v = idx_v[pl.ds(i * 16, 16)]
        m = (v >= lo) & (v < lo + PRES_PER_TILE)
        off = jnp.where(m, v - lo, 0)
        plsc.store_scatter(buf_v, [off], ones, mask=m)
        return 0

    lax.fori_loop(0, B // 16, scat_body, 0)
    pltpu.sync_copy(buf_v, pres_hbm.at[pl.ds(lo, PRES_PER_TILE)])


def _presence(start):
    return pl.kernel(
        _presence_body,
        out_type=jax.ShapeDtypeStruct((PV,), jnp.float32),
        mesh=_mk_mesh(),
        compiler_params=pltpu.CompilerParams(needs_layout_passes=False, use_tc_tiling_on_sc=False),
        scratch_types=[
            pltpu.VMEM((B,), jnp.int32),
            pltpu.VMEM((PRES_PER_TILE,), jnp.float32),
        ],
    )(start)


def _prefix_body(p_ref, r_ref):
    x = p_ref[:]
    ii = lax.broadcasted_iota(jnp.int32, (128, 128), 0)
    jj = lax.broadcasted_iota(jnp.int32, (128, 128), 1)
    u = (ii < jj).astype(jnp.float32)
    inrow_exc = jnp.dot(x, u, preferred_element_type=jnp.float32)
    rowtot = jnp.sum(x, axis=1, keepdims=True)
    t = rowtot
    sh = 1
    while sh < 8192:
        t = t + jnp.concatenate(
            [jnp.zeros((sh, 1), jnp.float32), t[:-sh]], axis=0)
        sh *= 2
    rowpref_exc = t - rowtot
    r_ref[:] = (inrow_exc + rowpref_exc).astype(jnp.int32)


def _ranks(presence):
    out = pl.pallas_call(
        _prefix_body,
        out_shape=jax.ShapeDtypeStruct((PV // 128, 128), jnp.int32),
    )(presence.reshape(PV // 128, 128))
    return out.reshape(PV)


def _compact_body(start_hbm, ranks_hbm, out_hbm, idx_v, rank_v, out_v, sem):
    lo = _wid() * ROWS_PER_TILE
    pltpu.sync_copy(start_hbm, idx_v)

    def gat_body(i, _):
        pltpu.async_copy(
            ranks_hbm.at[idx_v.at[pl.ds(i * 128, 128)]],
            rank_v.at[pl.ds(i * 128, 128)],
            sem,
        ).wait()
        return 0

    lax.fori_loop(0, B // 128, gat_body, 0)

    zeros = jnp.zeros((16,), jnp.int32)

    def zero_body(i, _):
        out_v[pl.ds(i * 16, 16)] = zeros
        return 0

    lax.fori_loop(0, ROWS_PER_TILE // 16, zero_body, 0)

    def filt_body(i, _):
        r = rank_v[pl.ds(i * 16, 16)]
        v = idx_v[pl.ds(i * 16, 16)]
        m = (r >= lo) & (r < lo + ROWS_PER_TILE)
        off = jnp.where(m, r - lo, 0)
        plsc.store_scatter(out_v, [off], v, mask=m)
        return 0

    lax.fori_loop(0, B // 16, filt_body, 0)
    pltpu.sync_copy(out_v, out_hbm.at[pl.ds(lo, ROWS_PER_TILE)])


def _compact(start, ranks):
    return pl.kernel(
        _compact_body,
        out_type=jax.ShapeDtypeStruct((B,), jnp.int32),
        mesh=_mk_mesh(),
        compiler_params=pltpu.CompilerParams(needs_layout_passes=False, use_tc_tiling_on_sc=False),
        scratch_types=[
            pltpu.VMEM((B,), jnp.int32),
            pltpu.VMEM((B,), jnp.int32),
            pltpu.VMEM((ROWS_PER_TILE,), jnp.int32),
            pltpu.SemaphoreType.DMA,
        ],
    )(start, ranks)


def _scores_body(idx_hbm, table_hbm, out_hbm, idx_v, g_v, out_v, sem):
    base = _wid() * ROWS_PER_TILE
    pltpu.sync_copy(idx_hbm.at[pl.ds(base, ROWS_PER_TILE)], idx_v)

    lanes = lax.iota(jnp.int32, 16)
    zeros = jnp.zeros((16,), jnp.float32)

    def row_body(r, _):
        pltpu.async_copy(table_hbm.at[idx_v.at[r]], g_v, sem).wait()
        s0 = g_v[0, pl.ds(0, 16)]
        s1 = g_v[0, pl.ds(16, 16)]
        s2 = g_v[0, pl.ds(32, 16)]
        s3 = g_v[0, pl.ds(48, 16)]

        def dot_at(t):
            p = s0 * g_v[t, pl.ds(0, 16)]
            p = p + s1 * g_v[t, pl.ds(16, 16)]
            p = p + s2 * g_v[t, pl.ds(32, 16)]
            p = p + s3 * g_v[t, pl.ds(48, 16)]
            return jnp.sum(p)

        for grp in range(3):
            def lane_body(l, acc):
                return jnp.where(lanes == l, dot_at(grp * 16 + l + 1), acc)
            out_v[r, pl.ds(grp * 16, 16)] = lax.fori_loop(
                0, 16, lane_body, zeros)

        acc = jnp.where(lanes == 0, dot_at(49), zeros)
        n0 = g_v[55, pl.ds(0, 16)]
        n1 = g_v[55, pl.ds(16, 16)]
        n2 = g_v[55, pl.ds(32, 16)]
        n3 = g_v[55, pl.ds(48, 16)]
        for k in range(KNEG):
            x0 = n0 * g_v[50 + k, pl.ds(0, 16)]
            x1 = n1 * g_v[50 + k, pl.ds(16, 16)]
            x2 = n2 * g_v[50 + k, pl.ds(32, 16)]
            x3 = n3 * g_v[50 + k, pl.ds(48, 16)]
            p = 1.0 / (1.0 + jnp.exp(-x0))
            p = p + 1.0 / (1.0 + jnp.exp(-x1))
            p = p + 1.0 / (1.0 + jnp.exp(-x2))
            p = p + 1.0 / (1.0 + jnp.exp(-x3))
            acc = jnp.where(lanes == 1 + k, jnp.sum(p), acc)
        out_v[r, pl.ds(48, 16)] = acc
        return 0

    lax.fori_loop(0, ROWS_PER_TILE, row_body, 0)
    pltpu.sync_copy(out_v, out_hbm.at[pl.ds(base, ROWS_PER_TILE)])


def _scores(idx_all, emb_table):
    return pl.kernel(
        _scores_body,
        out_type=jax.ShapeDtypeStruct((B, D), jnp.float32),
        mesh=_mk_mesh(),
        compiler_params=pltpu.CompilerParams(needs_layout_passes=False, use_tc_tiling_on_sc=False),
        scratch_types=[
            pltpu.VMEM((ROWS_PER_TILE, IDXW), jnp.int32),
            pltpu.VMEM((IDXW, D), jnp.float32),
            pltpu.VMEM((ROWS_PER_TILE, D), jnp.float32),
            pltpu.SemaphoreType.DMA,
        ],
    )(idx_all, emb_table)


def _loss_body(s_ref, o_ref):
    x = s_ref[:]
    col = lax.broadcasted_iota(jnp.int32, x.shape, 1)
    posm = col < (WL - 1)
    negm = (col >= (WL - 1)) & (col < (WL - 1 + KNEG))
    post = jnp.where(posm, jnp.log(jax.nn.sigmoid(x)), 0.0)
    negt = jnp.where(negm, jnp.log(x), 0.0)
    loss = -jnp.sum(post) + jnp.sum(negt)
    node_num = B // 8
    loss = loss / node_num / 8.0 / (WL - 1)
    o_ref[:] = jnp.reshape(loss, (1, 1))


def _loss(scores):
    return pl.pallas_call(
        _loss_body,
        out_shape=jax.ShapeDtypeStruct((1, 1), jnp.float32),
    )(scores)


def kernel(walks, sample_node, emb_table):
    walks = walks.astype(jnp.int32)
    sample_node = sample_node.astype(jnp.int32)
    start = walks[:, 0]
    presence = _presence(start)
    ranks = _ranks(presence)
    all_node = _compact(start, ranks)
    idx_all = jnp.concatenate([walks, sample_node, all_node[:, None]], axis=1)
    scores = _scores(idx_all, emb_table)
    return jnp.reshape(_loss(scores), ())


# speedup vs baseline: 2.2837x; 1.3251x over previous
import jax
import jax.numpy as jnp
from jax import lax
from jax.experimental import pallas as pl
from jax.experimental.pallas import tpu as pltpu
from jax.experimental.pallas import tpu_sc as plsc

NC = 2
NS = 16
NW = NC * NS
PV = 1 << 20
B = 16384
WL = 50
KNEG = 5
D = 64
IDXW = WL + KNEG + 1
ROWS_PER_TILE = B // NW
PRES_PER_TILE = PV // NW

def _mk_mesh():
    return plsc.VectorSubcoreMesh(core_axis_name="c", subcore_axis_name="s")


def _wid():
    return lax.axis_index("s") * NC + lax.axis_index("c")


def _presence_body(start_hbm, pres_hbm, idx_v, buf_v):
    lo = _wid() * PRES_PER_TILE
    pltpu.sync_copy(start_hbm, idx_v)
    zeros = jnp.zeros((16,), jnp.float32)
    ones = jnp.ones((16,), jnp.float32)

    def zero_body(i, _):
        buf_v[pl.ds(i * 16, 16)] = zeros
        return 0

    lax.fori_loop(0, PRES_PER_TILE // 16, zero_body, 0)

    def scat_body(i, _):
        v = idx_v[pl.ds(i * 16, 16)]
        m = (v >= lo) & (v < lo + PRES_PER_TILE)
        off = jnp.where(m, v - lo, 0)
        plsc.store_scatter(buf_v, [off], ones, mask=m)
        return 0

    lax.fori_loop(0, B // 16, scat_body, 0)
    pltpu.sync_copy(buf_v, pres_hbm.at[pl.ds(lo, PRES_PER_TILE)])


def _presence(start):
    return pl.kernel(
        _presence_body,
        out_type=jax.ShapeDtypeStruct((PV,), jnp.float32),
        mesh=_mk_mesh(),
        compiler_params=pltpu.CompilerParams(needs_layout_passes=False, use_tc_tiling_on_sc=False),
        scratch_types=[
            pltpu.VMEM((B,), jnp.int32),
            pltpu.VMEM((PRES_PER_TILE,), jnp.float32),
        ],
    )(start)


def _prefix_body(p_ref, r_ref):
    x = p_ref[:]
    ii = lax.broadcasted_iota(jnp.int32, (128, 128), 0)
    jj = lax.broadcasted_iota(jnp.int32, (128, 128), 1)
    u = (ii < jj).astype(jnp.float32)
    inrow_exc = jnp.dot(x, u, preferred_element_type=jnp.float32)
    rowtot = jnp.sum(x, axis=1, keepdims=True)
    t = rowtot
    sh = 1
    while sh < 8192:
        t = t + jnp.concatenate(
            [jnp.zeros((sh, 1), jnp.float32), t[:-sh]], axis=0)
        sh *= 2
    rowpref_exc = t - rowtot
    r_ref[:] = (inrow_exc + rowpref_exc).astype(jnp.int32)


def _ranks(presence):
    out = pl.pallas_call(
        _prefix_body,
        out_shape=jax.ShapeDtypeStruct((PV // 128, 128), jnp.int32),
    )(presence.reshape(PV // 128, 128))
    return out.reshape(PV)


def _compact_body(start_hbm, ranks_hbm, out_hbm, idx_v, rank_v, out_v, *gsems):
    lo = _wid() * ROWS_PER_TILE
    pltpu.sync_copy(start_hbm, idx_v)

    GW = 4

    def fire(c, s):
        pltpu.async_copy(
            ranks_hbm.at[idx_v.at[pl.ds(c * 128, 128)]],
            rank_v.at[pl.ds(c * 128, 128)],
            s,
        )

    def drain(c, s):
        pltpu.make_async_copy(
            ranks_hbm.at[idx_v.at[pl.ds(c * 128, 128)]],
            rank_v.at[pl.ds(c * 128, 128)],
            s,
        ).wait()

    for b in range(GW):
        fire(b, gsems[b])

    def gat_body(g, _):
        for b in range(GW):
            c = g * GW + b
            drain(c, gsems[b])
            nxt = c + GW

            @pl.when(nxt < B // 128)
            def _():
                fire(nxt, gsems[b])
        return 0

    lax.fori_loop(0, B // 128 // GW, gat_body, 0)

    zeros = jnp.zeros((16,), jnp.int32)

    def zero_body(i, _):
        out_v[pl.ds(i * 16, 16)] = zeros
        return 0

    lax.fori_loop(0, ROWS_PER_TILE // 16, zero_body, 0)

    def filt_body(i, _):
        r = rank_v[pl.ds(i * 16, 16)]
        v = idx_v[pl.ds(i * 16, 16)]
        m = (r >= lo) & (r < lo + ROWS_PER_TILE)
        off = jnp.where(m, r - lo, 0)
        plsc.store_scatter(out_v, [off], v, mask=m)
        return 0

    lax.fori_loop(0, B // 16, filt_body, 0)
    pltpu.sync_copy(out_v, out_hbm.at[pl.ds(lo, ROWS_PER_TILE)])


def _compact(start, ranks):
    return pl.kernel(
        _compact_body,
        out_type=jax.ShapeDtypeStruct((B,), jnp.int32),
        mesh=_mk_mesh(),
        compiler_params=pltpu.CompilerParams(needs_layout_passes=False, use_tc_tiling_on_sc=False),
        scratch_types=[
            pltpu.VMEM((B,), jnp.int32),
            pltpu.VMEM((B,), jnp.int32),
            pltpu.VMEM((ROWS_PER_TILE,), jnp.int32),
            pltpu.SemaphoreType.DMA,
            pltpu.SemaphoreType.DMA,
            pltpu.SemaphoreType.DMA,
            pltpu.SemaphoreType.DMA,
        ],
    )(start, ranks)


RING = 4


def _scores_body(idx_hbm, table_hbm, out_hbm, idx_v, out_v, *ring):
    gbufs, sems = ring[:RING], ring[RING:]
    base = _wid() * ROWS_PER_TILE
    pltpu.sync_copy(idx_hbm.at[pl.ds(base, ROWS_PER_TILE)], idx_v)

    lanes = lax.iota(jnp.int32, 16)
    zeros = jnp.zeros((16,), jnp.float32)

    def fire(r, gb, s):
        pltpu.async_copy(table_hbm.at[idx_v.at[r]], gb, s)

    def compute(r, g_v):
        s0 = g_v[0, pl.ds(0, 16)]
        s1 = g_v[0, pl.ds(16, 16)]
        s2 = g_v[0, pl.ds(32, 16)]
        s3 = g_v[0, pl.ds(48, 16)]

        def dot_at(t):
            p = s0 * g_v[t, pl.ds(0, 16)]
            p = p + s1 * g_v[t, pl.ds(16, 16)]
            p = p + s2 * g_v[t, pl.ds(32, 16)]
            p = p + s3 * g_v[t, pl.ds(48, 16)]
            return jnp.sum(p)

        for grp in range(3):
            def lane_body(l, acc):
                return jnp.where(lanes == l, dot_at(grp * 16 + l + 1), acc)
            out_v[r, pl.ds(grp * 16, 16)] = lax.fori_loop(
                0, 16, lane_body, zeros, unroll=8)

        acc = jnp.where(lanes == 0, dot_at(49), zeros)
        n0 = g_v[55, pl.ds(0, 16)]
        n1 = g_v[55, pl.ds(16, 16)]
        n2 = g_v[55, pl.ds(32, 16)]
        n3 = g_v[55, pl.ds(48, 16)]
        for k in range(KNEG):
            x0 = n0 * g_v[50 + k, pl.ds(0, 16)]
            x1 = n1 * g_v[50 + k, pl.ds(16, 16)]
            x2 = n2 * g_v[50 + k, pl.ds(32, 16)]
            x3 = n3 * g_v[50 + k, pl.ds(48, 16)]
            p = 1.0 / (1.0 + jnp.exp(-x0))
            p = p + 1.0 / (1.0 + jnp.exp(-x1))
            p = p + 1.0 / (1.0 + jnp.exp(-x2))
            p = p + 1.0 / (1.0 + jnp.exp(-x3))
            acc = jnp.where(lanes == 1 + k, jnp.sum(p), acc)
        out_v[r, pl.ds(48, 16)] = acc

    for b in range(RING):
        fire(b, gbufs[b], sems[b])

    def row_body(g, _):
        for b in range(RING):
            r = g * RING + b
            pltpu.make_async_copy(
                table_hbm.at[idx_v.at[r]], gbufs[b], sems[b]).wait()
            compute(r, gbufs[b])
            nxt = r + RING

            @pl.when(nxt < ROWS_PER_TILE)
            def _():
                fire(nxt, gbufs[b], sems[b])
        return 0

    lax.fori_loop(0, ROWS_PER_TILE // RING, row_body, 0)
    pltpu.sync_copy(out_v, out_hbm.at[pl.ds(base, ROWS_PER_TILE)])


def _scores(idx_all, emb_table):
    return pl.kernel(
        _scores_body,
        out_type=jax.ShapeDtypeStruct((B, D), jnp.float32),
        mesh=_mk_mesh(),
        compiler_params=pltpu.CompilerParams(needs_layout_passes=False, use_tc_tiling_on_sc=False),
        scratch_types=[
            pltpu.VMEM((ROWS_PER_TILE, IDXW), jnp.int32),
            pltpu.VMEM((ROWS_PER_TILE, D), jnp.float32),
        ]
        + [pltpu.VMEM((IDXW, D), jnp.float32) for _ in range(RING)]
        + [pltpu.SemaphoreType.DMA for _ in range(RING)],
    )(idx_all, emb_table)


def _loss_body(s_ref, o_ref):
    x = s_ref[:]
    col = lax.broadcasted_iota(jnp.int32, x.shape, 1)
    posm = col < (WL - 1)
    negm = (col >= (WL - 1)) & (col < (WL - 1 + KNEG))
    post = jnp.where(posm, jnp.log(jax.nn.sigmoid(x)), 0.0)
    negt = jnp.where(negm, jnp.log(x), 0.0)
    loss = -jnp.sum(post) + jnp.sum(negt)
    node_num = B // 8
    loss = loss / node_num / 8.0 / (WL - 1)
    o_ref[:] = jnp.reshape(loss, (1, 1))


def _loss(scores):
    return pl.pallas_call(
        _loss_body,
        out_shape=jax.ShapeDtypeStruct((1, 1), jnp.float32),
    )(scores)


def kernel(walks, sample_node, emb_table):
    walks = walks.astype(jnp.int32)
    sample_node = sample_node.astype(jnp.int32)
    start = walks[:, 0]
    presence = _presence(start)
    ranks = _ranks(presence)
    all_node = _compact(start, ranks)
    idx_all = jnp.concatenate([walks, sample_node, all_node[:, None]], axis=1)
    scores = _scores(idx_all, emb_table)
    return jnp.reshape(_loss(scores), ())


# speedup vs baseline: 2.3806x; 1.0424x over previous
import jax
import jax.numpy as jnp
from jax import lax
from jax.experimental import pallas as pl
from jax.experimental.pallas import tpu as pltpu
from jax.experimental.pallas import tpu_sc as plsc

NC = 2
NS = 16
NW = NC * NS
PV = 1 << 20
B = 16384
WL = 50
KNEG = 5
D = 64
IDXW = WL + KNEG + 1
ROWS_PER_TILE = B // NW
PRES_PER_TILE = PV // NW

def _mk_mesh():
    return plsc.VectorSubcoreMesh(core_axis_name="c", subcore_axis_name="s")


def _wid():
    return lax.axis_index("s") * NC + lax.axis_index("c")


def _presence_body(start_hbm, pres_hbm, idx_v, buf_v):
    lo = _wid() * PRES_PER_TILE
    pltpu.sync_copy(start_hbm, idx_v)
    zeros = jnp.zeros((16,), jnp.float32)
    ones = jnp.ones((16,), jnp.float32)

    def zero_body(i, _):
        buf_v[pl.ds(i * 16, 16)] = zeros
        return 0

    lax.fori_loop(0, PRES_PER_TILE // 16, zero_body, 0)

    def scat_body(i, _):
        v = idx_v[pl.ds(i * 16, 16)]
        m = (v >= lo) & (v < lo + PRES_PER_TILE)
        off = jnp.where(m, v - lo, 0)
        plsc.store_scatter(buf_v, [off], ones, mask=m)
        return 0

    lax.fori_loop(0, B // 16, scat_body, 0)
    pltpu.sync_copy(buf_v, pres_hbm.at[pl.ds(lo, PRES_PER_TILE)])


def _presence(start):
    return pl.kernel(
        _presence_body,
        out_type=jax.ShapeDtypeStruct((PV,), jnp.float32),
        mesh=_mk_mesh(),
        compiler_params=pltpu.CompilerParams(needs_layout_passes=False, use_tc_tiling_on_sc=False),
        scratch_types=[
            pltpu.VMEM((B,), jnp.int32),
            pltpu.VMEM((PRES_PER_TILE,), jnp.float32),
        ],
    )(start)


def _prefix_body(p_ref, r_ref):
    x = p_ref[:]
    ii = lax.broadcasted_iota(jnp.int32, (128, 128), 0)
    jj = lax.broadcasted_iota(jnp.int32, (128, 128), 1)
    u = (ii < jj).astype(jnp.float32)
    inrow_exc = jnp.dot(x, u, preferred_element_type=jnp.float32)
    rowtot = jnp.sum(x, axis=1, keepdims=True)
    t = rowtot
    sh = 1
    while sh < 8192:
        t = t + jnp.concatenate(
            [jnp.zeros((sh, 1), jnp.float32), t[:-sh]], axis=0)
        sh *= 2
    rowpref_exc = t - rowtot
    r_ref[:] = (inrow_exc + rowpref_exc).astype(jnp.int32)


def _ranks(presence):
    out = pl.pallas_call(
        _prefix_body,
        out_shape=jax.ShapeDtypeStruct((PV // 128, 128), jnp.int32),
    )(presence.reshape(PV // 128, 128))
    return out.reshape(PV)


def _compact_body(start_hbm, ranks_hbm, out_hbm, idx_v, rank_v, out_v, *gsems):
    lo = _wid() * ROWS_PER_TILE
    pltpu.sync_copy(start_hbm, idx_v)

    GW = 4

    def fire(c, s):
        pltpu.async_copy(
            ranks_hbm.at[idx_v.at[pl.ds(c * 128, 128)]],
            rank_v.at[pl.ds(c * 128, 128)],
            s,
        )

    def drain(c, s):
        pltpu.make_async_copy(
            ranks_hbm.at[idx_v.at[pl.ds(c * 128, 128)]],
            rank_v.at[pl.ds(c * 128, 128)],
            s,
        ).wait()

    for b in range(GW):
        fire(b, gsems[b])

    def gat_body(g, _):
        for b in range(GW):
            c = g * GW + b
            drain(c, gsems[b])
            nxt = c + GW

            @pl.when(nxt < B // 128)
            def _():
                fire(nxt, gsems[b])
        return 0

    lax.fori_loop(0, B // 128 // GW, gat_body, 0)

    zeros = jnp.zeros((16,), jnp.int32)

    def zero_body(i, _):
        out_v[pl.ds(i * 16, 16)] = zeros
        return 0

    lax.fori_loop(0, ROWS_PER_TILE // 16, zero_body, 0)

    def filt_body(i, _):
        r = rank_v[pl.ds(i * 16, 16)]
        v = idx_v[pl.ds(i * 16, 16)]
        m = (r >= lo) & (r < lo + ROWS_PER_TILE)
        off = jnp.where(m, r - lo, 0)
        plsc.store_scatter(out_v, [off], v, mask=m)
        return 0

    lax.fori_loop(0, B // 16, filt_body, 0)
    pltpu.sync_copy(out_v, out_hbm.at[pl.ds(lo, ROWS_PER_TILE)])


def _compact(start, ranks):
    return pl.kernel(
        _compact_body,
        out_type=jax.ShapeDtypeStruct((B,), jnp.int32),
        mesh=_mk_mesh(),
        compiler_params=pltpu.CompilerParams(needs_layout_passes=False, use_tc_tiling_on_sc=False),
        scratch_types=[
            pltpu.VMEM((B,), jnp.int32),
            pltpu.VMEM((B,), jnp.int32),
            pltpu.VMEM((ROWS_PER_TILE,), jnp.int32),
            pltpu.SemaphoreType.DMA,
            pltpu.SemaphoreType.DMA,
            pltpu.SemaphoreType.DMA,
            pltpu.SemaphoreType.DMA,
        ],
    )(start, ranks)


RING = 4
GB = 2
NGT = ROWS_PER_TILE // GB


def _scores_body(idx_hbm, table_hbm, out_hbm, idx_v, out_v, *ring):
    gbufs, sems = ring[:RING], ring[RING:]
    wid = _wid()
    base = wid * ROWS_PER_TILE
    pltpu.sync_copy(idx_hbm.at[pl.ds(wid * NGT, NGT)], idx_v)

    lanes = lax.iota(jnp.int32, 16)
    zeros = jnp.zeros((16,), jnp.float32)

    def fire(g, gb, s):
        pltpu.async_copy(table_hbm.at[idx_v.at[g]], gb, s)

    def compute(r, u, g_v):
        o = u * IDXW
        s0 = g_v[o, pl.ds(0, 16)]
        s1 = g_v[o, pl.ds(16, 16)]
        s2 = g_v[o, pl.ds(32, 16)]
        s3 = g_v[o, pl.ds(48, 16)]

        def dot_at(t):
            p = s0 * g_v[o + t, pl.ds(0, 16)]
            p = p + s1 * g_v[o + t, pl.ds(16, 16)]
            p = p + s2 * g_v[o + t, pl.ds(32, 16)]
            p = p + s3 * g_v[o + t, pl.ds(48, 16)]
            return jnp.sum(p)

        for grp in range(3):
            def lane_body(l, acc):
                return jnp.where(lanes == l, dot_at(grp * 16 + l + 1), acc)
            out_v[r, pl.ds(grp * 16, 16)] = lax.fori_loop(
                0, 16, lane_body, zeros, unroll=8)

        acc = jnp.where(lanes == 0, dot_at(49), zeros)
        n0 = g_v[o + 55, pl.ds(0, 16)]
        n1 = g_v[o + 55, pl.ds(16, 16)]
        n2 = g_v[o + 55, pl.ds(32, 16)]
        n3 = g_v[o + 55, pl.ds(48, 16)]
        for k in range(KNEG):
            x0 = n0 * g_v[o + 50 + k, pl.ds(0, 16)]
            x1 = n1 * g_v[o + 50 + k, pl.ds(16, 16)]
            x2 = n2 * g_v[o + 50 + k, pl.ds(32, 16)]
            x3 = n3 * g_v[o + 50 + k, pl.ds(48, 16)]
            p = 1.0 / (1.0 + jnp.exp(-x0))
            p = p + 1.0 / (1.0 + jnp.exp(-x1))
            p = p + 1.0 / (1.0 + jnp.exp(-x2))
            p = p + 1.0 / (1.0 + jnp.exp(-x3))
            acc = jnp.where(lanes == 1 + k, jnp.sum(p), acc)
        out_v[r, pl.ds(48, 16)] = acc

    for b in range(RING):
        fire(b, gbufs[b], sems[b])

    def row_body(i, _):
        for b in range(RING):
            g = i * RING + b
            pltpu.make_async_copy(
                table_hbm.at[idx_v.at[g]], gbufs[b], sems[b]).wait()

            def u_body(u, _):
                compute(g * GB + u, u, gbufs[b])
                return 0

            lax.fori_loop(0, GB, u_body, 0)
            nxt = g + RING

            @pl.when(nxt < NGT)
            def _():
                fire(nxt, gbufs[b], sems[b])
        return 0

    lax.fori_loop(0, NGT // RING, row_body, 0)
    pltpu.sync_copy(out_v, out_hbm.at[pl.ds(base, ROWS_PER_TILE)])


def _scores(idx_all, emb_table):
    return pl.kernel(
        _scores_body,
        out_type=jax.ShapeDtypeStruct((B, D), jnp.float32),
        mesh=_mk_mesh(),
        compiler_params=pltpu.CompilerParams(needs_layout_passes=False, use_tc_tiling_on_sc=False),
        scratch_types=[
            pltpu.VMEM((NGT, GB * IDXW), jnp.int32),
            pltpu.VMEM((ROWS_PER_TILE, D), jnp.float32),
        ]
        + [pltpu.VMEM((GB * IDXW, D), jnp.float32) for _ in range(RING)]
        + [pltpu.SemaphoreType.DMA for _ in range(RING)],
    )(idx_all.reshape(B // GB, GB * IDXW), emb_table)


def _loss_body(s_ref, o_ref):
    x = s_ref[:]
    col = lax.broadcasted_iota(jnp.int32, x.shape, 1)
    posm = col < (WL - 1)
    negm = (col >= (WL - 1)) & (col < (WL - 1 + KNEG))
    post = jnp.where(posm, jnp.log(jax.nn.sigmoid(x)), 0.0)
    negt = jnp.where(negm, jnp.log(x), 0.0)
    loss = -jnp.sum(post) + jnp.sum(negt)
    node_num = B // 8
    loss = loss / node_num / 8.0 / (WL - 1)
    o_ref[:] = jnp.reshape(loss, (1, 1))


def _loss(scores):
    return pl.pallas_call(
        _loss_body,
        out_shape=jax.ShapeDtypeStruct((1, 1), jnp.float32),
    )(scores)


def kernel(walks, sample_node, emb_table):
    walks = walks.astype(jnp.int32)
    sample_node = sample_node.astype(jnp.int32)
    start = walks[:, 0]
    presence = _presence(start)
    ranks = _ranks(presence)
    all_node = _compact(start, ranks)
    idx_all = jnp.concatenate([walks, sample_node, all_node[:, None]], axis=1)
    scores = _scores(idx_all, emb_table)
    return jnp.reshape(_loss(scores), ())
